# Initial kernel scaffold; baseline (speedup 1.0000x reference)
import jax
import jax.numpy as jnp
from jax.experimental import pallas as pl


def kernel(x, W, b):
    raise NotImplementedError("write your pallas kernel here")


# speedup vs baseline: 1.4953x; 1.4953x over previous
import functools

import jax
import jax.numpy as jnp
from jax.experimental import pallas as pl
from jax.experimental.pallas import tpu as pltpu

D_MODEL_ = 2048
N_EXP_ = 64
BM_ = 512


def _gate_body(x_ref, wt_ref, b_ref, w_out_ref, i_out_ref):
    x_blk = x_ref[...]
    logits = jnp.dot(x_blk, wt_ref[...], preferred_element_type=jnp.float32)
    logits = logits + b_ref[...]

    iota = jax.lax.broadcasted_iota(jnp.int32, logits.shape, 1)
    neg_inf = jnp.float32(-jnp.inf)

    m1 = jnp.max(logits, axis=1, keepdims=True)
    i1 = jnp.min(jnp.where(logits == m1, iota, N_EXP_), axis=1, keepdims=True)
    masked = jnp.where(iota == i1, neg_inf, logits)
    m2 = jnp.max(masked, axis=1, keepdims=True)
    i2 = jnp.min(jnp.where(masked == m2, iota, N_EXP_), axis=1, keepdims=True)

    e2 = jnp.exp(m2 - m1)
    denom = 1.0 + e2
    w1 = 1.0 / denom
    w2 = e2 / denom

    w_out_ref[...] = jnp.concatenate([w1, w2], axis=1)
    i_out_ref[...] = jnp.concatenate([i1, i2], axis=1)


@functools.partial(jax.jit, static_argnums=())
def kernel(x, W, b):
    batch, seq, d_model = x.shape
    rows = batch * seq
    x_flat = x.reshape(rows, d_model)
    wt = W.T
    b2 = b.reshape(1, N_EXP_)

    grid = (rows // BM_,)
    w_out, i_out = pl.pallas_call(
        _gate_body,
        grid=grid,
        in_specs=[
            pl.BlockSpec((BM_, d_model), lambda i: (i, 0)),
            pl.BlockSpec((d_model, N_EXP_), lambda i: (0, 0)),
            pl.BlockSpec((1, N_EXP_), lambda i: (0, 0)),
        ],
        out_specs=[
            pl.BlockSpec((BM_, 2), lambda i: (i, 0)),
            pl.BlockSpec((BM_, 2), lambda i: (i, 0)),
        ],
        out_shape=[
            jax.ShapeDtypeStruct((rows, 2), jnp.float32),
            jax.ShapeDtypeStruct((rows, 2), jnp.int32),
        ],
        compiler_params=pltpu.CompilerParams(
            dimension_semantics=("arbitrary",),
        ),
    )(x_flat, wt, b2)
    return (w_out, i_out)


# speedup vs baseline: 1.7430x; 1.1657x over previous
import functools

import jax
import jax.numpy as jnp
from jax.experimental import pallas as pl
from jax.experimental.pallas import tpu as pltpu

D_MODEL_ = 2048
N_EXP_ = 64
BM_ = 1024


def _gate_body(x_ref, wt_ref, b_ref, w_out_ref, i_out_ref):
    x_blk = x_ref[...]
    logits = jnp.dot(x_blk, wt_ref[...], preferred_element_type=jnp.float32)
    logits = logits + b_ref[...]

    iota = jax.lax.broadcasted_iota(jnp.int32, logits.shape, 1)
    neg_inf = jnp.float32(-jnp.inf)

    m1 = jnp.max(logits, axis=1, keepdims=True)
    i1 = jnp.min(jnp.where(logits == m1, iota, N_EXP_), axis=1, keepdims=True)
    masked = jnp.where(iota == i1, neg_inf, logits)
    m2 = jnp.max(masked, axis=1, keepdims=True)
    i2 = jnp.min(jnp.where(masked == m2, iota, N_EXP_), axis=1, keepdims=True)

    e2 = jnp.exp(m2 - m1)
    denom = 1.0 + e2
    w1 = 1.0 / denom
    w2 = e2 / denom

    w_out_ref[...] = jnp.concatenate([w1, w2], axis=1)
    i_out_ref[...] = jnp.concatenate([i1, i2], axis=1)


@functools.partial(jax.jit, static_argnums=())
def kernel(x, W, b):
    batch, seq, d_model = x.shape
    rows = batch * seq
    x_flat = x.reshape(rows, d_model)
    wt = W.T
    b2 = b.reshape(1, N_EXP_)

    grid = (rows // BM_,)
    w_out, i_out = pl.pallas_call(
        _gate_body,
        grid=grid,
        in_specs=[
            pl.BlockSpec((BM_, d_model), lambda i: (i, 0)),
            pl.BlockSpec((d_model, N_EXP_), lambda i: (0, 0)),
            pl.BlockSpec((1, N_EXP_), lambda i: (0, 0)),
        ],
        out_specs=[
            pl.BlockSpec((BM_, 2), lambda i: (i, 0)),
            pl.BlockSpec((BM_, 2), lambda i: (i, 0)),
        ],
        out_shape=[
            jax.ShapeDtypeStruct((rows, 2), jnp.float32),
            jax.ShapeDtypeStruct((rows, 2), jnp.int32),
        ],
        compiler_params=pltpu.CompilerParams(
            dimension_semantics=("arbitrary",),
        ),
    )(x_flat, wt, b2)
    return (w_out, i_out)


# speedup vs baseline: 1.8171x; 1.0425x over previous
import functools

import jax
import jax.numpy as jnp
from jax.experimental import pallas as pl
from jax.experimental.pallas import tpu as pltpu

D_MODEL_ = 2048
N_EXP_ = 64
BM_ = 2048


def _gate_body(x_ref, wt_ref, b_ref, w_out_ref, i_out_ref):
    x_blk = x_ref[...]
    logits = jnp.dot(x_blk, wt_ref[...], preferred_element_type=jnp.float32)
    logits = logits + b_ref[...]

    iota = jax.lax.broadcasted_iota(jnp.int32, logits.shape, 1)
    neg_inf = jnp.float32(-jnp.inf)

    m1 = jnp.max(logits, axis=1, keepdims=True)
    i1 = jnp.min(jnp.where(logits == m1, iota, N_EXP_), axis=1, keepdims=True)
    masked = jnp.where(iota == i1, neg_inf, logits)
    m2 = jnp.max(masked, axis=1, keepdims=True)
    i2 = jnp.min(jnp.where(masked == m2, iota, N_EXP_), axis=1, keepdims=True)

    e2 = jnp.exp(m2 - m1)
    denom = 1.0 + e2
    w1 = 1.0 / denom
    w2 = e2 / denom

    w_out_ref[...] = jnp.concatenate([w1, w2], axis=1)
    i_out_ref[...] = jnp.concatenate([i1, i2], axis=1)


@functools.partial(jax.jit, static_argnums=())
def kernel(x, W, b):
    batch, seq, d_model = x.shape
    rows = batch * seq
    x_flat = x.reshape(rows, d_model)
    wt = W.T
    b2 = b.reshape(1, N_EXP_)

    grid = (rows // BM_,)
    w_out, i_out = pl.pallas_call(
        _gate_body,
        grid=grid,
        in_specs=[
            pl.BlockSpec((BM_, d_model), lambda i: (i, 0)),
            pl.BlockSpec((d_model, N_EXP_), lambda i: (0, 0)),
            pl.BlockSpec((1, N_EXP_), lambda i: (0, 0)),
        ],
        out_specs=[
            pl.BlockSpec((BM_, 2), lambda i: (i, 0)),
            pl.BlockSpec((BM_, 2), lambda i: (i, 0)),
        ],
        out_shape=[
            jax.ShapeDtypeStruct((rows, 2), jnp.float32),
            jax.ShapeDtypeStruct((rows, 2), jnp.int32),
        ],
        compiler_params=pltpu.CompilerParams(
            dimension_semantics=("arbitrary",),
        ),
    )(x_flat, wt, b2)
    return (w_out, i_out)
